# speedup vs baseline: 7.1256x; 7.1256x over previous
import jax
import jax.numpy as jnp
from jax.experimental import pallas as pl
from jax.experimental.pallas import tpu as pltpu

NUM_SEGMENTS = 1024
ROWS = 50000
FEAT = 256
CHUNK = 2000
NCHUNK = ROWS // CHUNK


def _body(b_ref, x_ref, o_ref, cnt_ref):
    i = pl.program_id(0)
    bvec = b_ref[0, 0, :]
    gids = jax.lax.broadcasted_iota(jnp.int32, (NUM_SEGMENTS, CHUNK), 0)
    onehot = (gids == bvec[None, :]).astype(jnp.bfloat16)

    x = x_ref[...]
    xhi = x.astype(jnp.bfloat16)
    xlo = (x - xhi.astype(jnp.float32)).astype(jnp.bfloat16)
    psum = (
        jax.lax.dot(onehot, xhi, preferred_element_type=jnp.float32)
        + jax.lax.dot(onehot, xlo, preferred_element_type=jnp.float32)
    )
    pcnt = jnp.sum(onehot.astype(jnp.float32), axis=1, keepdims=True)

    @pl.when(i == 0)
    def _():
        o_ref[...] = psum
        cnt_ref[...] = pcnt

    @pl.when(i > 0)
    def _():
        o_ref[...] += psum
        cnt_ref[...] += pcnt

    @pl.when(i == NCHUNK - 1)
    def _():
        o_ref[...] = o_ref[...] / jnp.maximum(cnt_ref[...], 1.0)


def kernel(x, batch):
    b3 = batch.astype(jnp.int32).reshape(NCHUNK, 1, CHUNK)
    return pl.pallas_call(
        _body,
        grid=(NCHUNK,),
        in_specs=[
            pl.BlockSpec((1, 1, CHUNK), lambda i: (i, 0, 0)),
            pl.BlockSpec((CHUNK, FEAT), lambda i: (i, 0)),
        ],
        out_specs=pl.BlockSpec((NUM_SEGMENTS, FEAT), lambda i: (0, 0)),
        out_shape=jax.ShapeDtypeStruct((NUM_SEGMENTS, FEAT), jnp.float32),
        scratch_shapes=[pltpu.VMEM((NUM_SEGMENTS, 1), jnp.float32)],
    )(b3, x)


# speedup vs baseline: 10.3925x; 1.4585x over previous
import jax
import jax.numpy as jnp
from jax.experimental import pallas as pl
from jax.experimental.pallas import tpu as pltpu

NUM_SEGMENTS = 1024
ROWS = 50000
FEAT = 256
CHUNK = 2000
NCHUNK = ROWS // CHUNK


def _body(b_ref, x_ref, o_ref, cnt_ref):
    i = pl.program_id(0)
    bvec = b_ref[0, 0, :]
    gids = jax.lax.broadcasted_iota(jnp.int32, (NUM_SEGMENTS, CHUNK), 0)
    onehot = (gids == bvec[None, :]).astype(jnp.bfloat16)

    xhi = x_ref[...].astype(jnp.bfloat16)
    psum = jax.lax.dot(onehot, xhi, preferred_element_type=jnp.float32)
    pcnt = jnp.sum(onehot.astype(jnp.float32), axis=1, keepdims=True)

    @pl.when(i == 0)
    def _():
        o_ref[...] = psum
        cnt_ref[...] = pcnt

    @pl.when(i > 0)
    def _():
        o_ref[...] += psum
        cnt_ref[...] += pcnt

    @pl.when(i == NCHUNK - 1)
    def _():
        o_ref[...] = o_ref[...] / jnp.maximum(cnt_ref[...], 1.0)


def kernel(x, batch):
    b3 = batch.astype(jnp.int32).reshape(NCHUNK, 1, CHUNK)
    return pl.pallas_call(
        _body,
        grid=(NCHUNK,),
        in_specs=[
            pl.BlockSpec((1, 1, CHUNK), lambda i: (i, 0, 0)),
            pl.BlockSpec((CHUNK, FEAT), lambda i: (i, 0)),
        ],
        out_specs=pl.BlockSpec((NUM_SEGMENTS, FEAT), lambda i: (0, 0)),
        out_shape=jax.ShapeDtypeStruct((NUM_SEGMENTS, FEAT), jnp.float32),
        scratch_shapes=[pltpu.VMEM((NUM_SEGMENTS, 1), jnp.float32)],
    )(b3, x)
